# baseline (speedup 1.0000x reference)
import functools

import jax
import jax.numpy as jnp
from jax import lax
from jax.experimental import pallas as pl
from jax.experimental.pallas import tpu as pltpu
from jax.experimental.pallas import tpu_sc as plsc

N = 50000
E = 800000
RW = 36
NP = 50176
ER = E // 128
ERH = ER // 2
CH = 1024
NCH = 25
PER_SC = NP * RW
PER_TILE = PER_SC // 16
ALLOC_W = 1812480
PER_TILE_Z = ALLOC_W // 16
ZB = 1920
CR = CH // 128
BN = 512
GRID = NP // BN


def _sc_hist(ei_hbm, code_hbm, out_hbm,
             srcv, dstv, codes_v, addr_v, val_v, zbuf,
             code_sh, counts_sh, sem_g, sem_a):
    c = lax.axis_index("c")
    s = lax.axis_index("s")

    def zfill(i, _):
        zbuf[pl.ds(i * 16, 16)] = jnp.zeros((16,), jnp.int32)
        return 0
    lax.fori_loop(0, ZB // 16, zfill, 0)

    zbase = s * PER_TILE_Z
    def zcp(i, _):
        pltpu.sync_copy(zbuf, counts_sh.at[pl.ds(zbase + i * ZB, ZB)])
        return 0
    lax.fori_loop(0, PER_TILE_Z // ZB, zcp, 0)

    @pl.when(s == 0)
    def _():
        pltpu.sync_copy(code_hbm, code_sh)
    plsc.subcore_barrier()

    def _drain_adds():
        for j in range(16):
            pltpu.make_async_copy(val_v.at[j], counts_sh.at[addr_v.at[j]],
                                  sem_a).wait()

    rstart = c * ERH + (s * ERH) // 16
    rend = c * ERH + ((s + 1) * ERH) // 16
    def chunk(g, _):
        rb = rstart + g * CR
        rb_c = jnp.minimum(rb, ER - CR)
        oc = rb_c * 128
        pltpu.sync_copy(ei_hbm.at[pl.ds(oc, CH)], srcv)
        pltpu.sync_copy(ei_hbm.at[pl.ds(E + oc, CH)], dstv)
        gcps = [pltpu.async_copy(code_sh.at[srcv.at[pl.ds(j * 128, 128)]],
                                 codes_v.at[pl.ds(j * 128, 128)], sem_g)
                for j in range(CR)]
        for cp in gcps:
            cp.wait()
        @pl.when(g > 0)
        def _():
            _drain_adds()
        for i in range(CH // 16):
            j, k = i // 8, (i % 8) * 16
            rg = rb_c + j
            pen = jnp.where((rg >= rb) & (rg < rend), 0, 10000000)
            cv = codes_v[pl.ds(j * 128 + k, 16)]
            dv = dstv[pl.ds(j * 128 + k, 16)]
            p = lax.shift_right_logical(cv, 7)
            dr = lax.bitwise_and(cv, 127)
            rowt = jnp.minimum(dv + pen, NP)
            base = rowt * RW
            pin = p < 36
            din = dr < 36
            addr_v[j, pl.ds(k, 16)] = base + jnp.where(pin, p, p - 36)
            addr_v[j + 8, pl.ds(k, 16)] = base + jnp.where(din, dr, dr - 36)
            val_v[j, pl.ds(k, 16)] = jnp.where(pin, 1, 256)
            val_v[j + 8, pl.ds(k, 16)] = jnp.where(din, 65536, 16777216)
        for j in range(16):
            pltpu.async_copy(val_v.at[j], counts_sh.at[addr_v.at[j]], sem_a,
                             add=True)
        return 0
    lax.fori_loop(0, NCH, chunk, 0)
    _drain_adds()
    plsc.subcore_barrier()

    gout = c * PER_SC + s * PER_TILE
    pltpu.sync_copy(counts_sh.at[pl.ds(s * PER_TILE, PER_TILE)],
                    out_hbm.at[pl.ds(gout, PER_TILE)])


_sc_hist_call = functools.partial(
    pl.kernel,
    mesh=plsc.VectorSubcoreMesh(core_axis_name="c", subcore_axis_name="s"),
    compiler_params=pltpu.CompilerParams(needs_layout_passes=False),
    out_type=jax.ShapeDtypeStruct((2 * NP * RW,), jnp.int32),
    scratch_types=[
        pltpu.VMEM((CH,), jnp.int32),
        pltpu.VMEM((CH,), jnp.int32),
        pltpu.VMEM((CH,), jnp.int32),
        pltpu.VMEM((16, 128), jnp.int32),
        pltpu.VMEM((16, 128), jnp.int32),
        pltpu.VMEM((ZB,), jnp.int32),
        pltpu.VMEM_SHARED((N,), jnp.int32),
        pltpu.VMEM_SHARED((ALLOC_W,), jnp.int32),
        pltpu.SemaphoreType.DMA,
        pltpu.SemaphoreType.DMA,
    ],
)(_sc_hist)


def _fold(pt_ref, dt_ref, wl_ref, wr_ref,
          wcp_ref, wcd_ref, wp_ref, wd_ref):
    pt = pt_ref[:]
    dt = dt_ref[:]
    wl = wl_ref[:]
    wr = wr_ref[:]
    wcp_ref[:] = jnp.dot(pt, wl[0:32, :], preferred_element_type=jnp.float32)
    wcd_ref[:] = jnp.dot(dt, wl[32:64, :], preferred_element_type=jnp.float32)
    wp_ref[:] = jnp.dot(pt, wr[0:32, :], preferred_element_type=jnp.float32)
    wd_ref[:] = jnp.dot(dt, wr[32:64, :], preferred_element_type=jnp.float32)


def _fold_call(pt72, dt72, W_l, W_r):
    w = jax.ShapeDtypeStruct((72, 128), jnp.float32)
    return pl.pallas_call(
        _fold, out_shape=[w, w, w, w],
    )(pt72, dt72, W_l, W_r)


def _main(cnt_ref, p_ref, d_ref, bi_ref, wcp_ref, wcd_ref, wp_ref, wd_ref,
          bt_ref, b_ref, o_ref):
    w0 = cnt_ref[0]
    w1 = cnt_ref[1]
    m8 = 255
    pc = jnp.concatenate(
        [(lax.bitwise_and(w0, m8)
          + lax.bitwise_and(w1, m8)).astype(jnp.float32),
         (lax.bitwise_and(lax.shift_right_logical(w0, 8), m8)
          + lax.bitwise_and(lax.shift_right_logical(w1, 8), m8)
          ).astype(jnp.float32)], axis=1)
    dc = jnp.concatenate(
        [(lax.bitwise_and(lax.shift_right_logical(w0, 16), m8)
          + lax.bitwise_and(lax.shift_right_logical(w1, 16), m8)
          ).astype(jnp.float32),
         (lax.shift_right_logical(w0, 24)
          + lax.shift_right_logical(w1, 24)).astype(jnp.float32)], axis=1)
    deg = jnp.sum(pc, axis=1, keepdims=True)
    inv = 1.0 / jnp.maximum(deg, 1.0)
    iot = lax.broadcasted_iota(jnp.int32, (BN, 72), 1)
    ohp = (p_ref[:] == iot).astype(jnp.float32)
    ohd = (d_ref[:] == iot).astype(jnp.float32)
    ohb = (bi_ref[:] == iot).astype(jnp.float32)
    left = (jnp.dot(pc * inv, wcp_ref[:], preferred_element_type=jnp.float32)
            + jnp.dot(dc * inv, wcd_ref[:], preferred_element_type=jnp.float32)
            + jnp.dot(ohp, wp_ref[:], preferred_element_type=jnp.float32)
            + jnp.dot(ohd, wd_ref[:], preferred_element_type=jnp.float32)
            + b_ref[:])
    ss = jnp.sum(left * left, axis=1, keepdims=True)
    left = left * jnp.minimum(lax.rsqrt(ss), 1e12)
    left = jnp.where(left > 0, left, 0.2 * left)
    o_ref[:, 0:128] = left
    o_ref[:, 128:160] = jnp.dot(ohb, bt_ref[:],
                                preferred_element_type=jnp.float32)


def _main_call(counts, p2, d2, b2, wcp, wcd, wp, wd, bt72, bias):
    full = lambda shp: pl.BlockSpec(shp, lambda i: (0, 0))
    return pl.pallas_call(
        _main,
        grid=(GRID,),
        in_specs=[
            pl.BlockSpec((2, BN, RW), lambda i: (0, i, 0)),
            pl.BlockSpec((BN, 1), lambda i: (i, 0)),
            pl.BlockSpec((BN, 1), lambda i: (i, 0)),
            pl.BlockSpec((BN, 1), lambda i: (i, 0)),
            full((72, 128)),
            full((72, 128)),
            full((72, 128)),
            full((72, 128)),
            full((72, 32)),
            full((1, 128)),
        ],
        out_specs=pl.BlockSpec((BN, 160), lambda i: (i, 0)),
        out_shape=jax.ShapeDtypeStruct((N, 160), jnp.float32),
    )(counts, p2, d2, b2, wcp, wcd, wp, wd, bt72, bias)


def _pad_ids(a):
    return jnp.pad(a, (0, NP - N)).reshape(NP, 1)


def kernel(x, beat_info, edge_index, pitch_table, beat_table, dur_table,
           W_l, W_r, b):
    pitch = x[:, 2]
    dur = x[:, 3]
    code = pitch * 128 + dur
    ei2 = edge_index.reshape(2 * E)

    counts = _sc_hist_call(ei2, code).reshape(2, NP, RW)

    p2 = _pad_ids(pitch)
    d2 = _pad_ids(dur)
    bi2 = _pad_ids(beat_info)
    pt72 = jnp.pad(pitch_table[:66], ((0, 6), (0, 0)))
    dt72 = jnp.pad(dur_table, ((0, 6), (0, 0)))
    bt72 = jnp.pad(beat_table, ((0, 6), (0, 0)))
    wcp, wcd, wp, wd = _fold_call(pt72, dt72, W_l, W_r)

    return _main_call(counts, p2, d2, bi2, wcp, wcd, wp, wd, bt72,
                      b.reshape(1, 128))


# speedup vs baseline: 1.0017x; 1.0017x over previous
import functools

import jax
import jax.numpy as jnp
from jax import lax
from jax.experimental import pallas as pl
from jax.experimental.pallas import tpu as pltpu
from jax.experimental.pallas import tpu_sc as plsc

N = 50000
E = 800000
RW = 36
NP = 50176
ER = E // 128
ERH = ER // 2
CH = 1024
NCH = 25
PER_SC = NP * RW
PER_TILE = PER_SC // 16
ALLOC_W = 1812480
PER_TILE_Z = ALLOC_W // 16
ZB = 1920
CR = CH // 128
BN = 512
GRID = NP // BN


def _sc_hist(ei_hbm, code_hbm, out_hbm,
             srcv, dstv, codes_v, addr_v, val_v, zbuf,
             code_sh, counts_sh, sem_g, sem_a):
    c = lax.axis_index("c")
    s = lax.axis_index("s")

    def zfill(i, _):
        zbuf[pl.ds(i * 16, 16)] = jnp.zeros((16,), jnp.int32)
        return 0
    lax.fori_loop(0, ZB // 16, zfill, 0)

    zbase = s * PER_TILE_Z
    def zcp(i, _):
        pltpu.sync_copy(zbuf, counts_sh.at[pl.ds(zbase + i * ZB, ZB)])
        return 0
    lax.fori_loop(0, PER_TILE_Z // ZB, zcp, 0)

    @pl.when(s == 0)
    def _():
        pltpu.sync_copy(code_hbm, code_sh)
    plsc.subcore_barrier()

    def _drain_adds():
        for j in range(16):
            pltpu.make_async_copy(val_v.at[j], counts_sh.at[addr_v.at[j]],
                                  sem_a).wait()

    rstart = c * ERH + (s * ERH) // 16
    rend = c * ERH + ((s + 1) * ERH) // 16
    def chunk(g, _):
        rb = rstart + g * CR
        rb_c = jnp.minimum(rb, ER - CR)
        oc = rb_c * 128
        pltpu.sync_copy(ei_hbm.at[pl.ds(oc, CH)], srcv)
        pltpu.sync_copy(ei_hbm.at[pl.ds(E + oc, CH)], dstv)
        gcps = [pltpu.async_copy(code_sh.at[srcv.at[pl.ds(j * 128, 128)]],
                                 codes_v.at[pl.ds(j * 128, 128)], sem_g)
                for j in range(CR)]
        for cp in gcps:
            cp.wait()
        @pl.when(g > 0)
        def _():
            _drain_adds()
        for i in range(CH // 16):
            j, k = i // 8, (i % 8) * 16
            rg = rb_c + j
            pen = jnp.where((rg >= rb) & (rg < rend), 0, 10000000)
            cv = codes_v[pl.ds(j * 128 + k, 16)]
            dv = dstv[pl.ds(j * 128 + k, 16)]
            p = lax.shift_right_logical(cv, 7)
            dr = lax.bitwise_and(cv, 127)
            rowt = jnp.minimum(dv + pen, NP)
            base = rowt * RW
            pin = p < 36
            din = dr < 36
            addr_v[j, pl.ds(k, 16)] = base + jnp.where(pin, p, p - 36)
            addr_v[j + 8, pl.ds(k, 16)] = base + jnp.where(din, dr, dr - 36)
            val_v[j, pl.ds(k, 16)] = jnp.where(pin, 1, 256)
            val_v[j + 8, pl.ds(k, 16)] = jnp.where(din, 65536, 16777216)
        for j in range(16):
            pltpu.async_copy(val_v.at[j], counts_sh.at[addr_v.at[j]], sem_a,
                             add=True)
        return 0
    lax.fori_loop(0, NCH, chunk, 0)
    _drain_adds()
    plsc.subcore_barrier()

    gout = c * PER_SC + s * PER_TILE
    pltpu.sync_copy(counts_sh.at[pl.ds(s * PER_TILE, PER_TILE)],
                    out_hbm.at[pl.ds(gout, PER_TILE)])


_sc_hist_call = functools.partial(
    pl.kernel,
    mesh=plsc.VectorSubcoreMesh(core_axis_name="c", subcore_axis_name="s"),
    compiler_params=pltpu.CompilerParams(needs_layout_passes=False),
    out_type=jax.ShapeDtypeStruct((2 * NP * RW,), jnp.int32),
    scratch_types=[
        pltpu.VMEM((CH,), jnp.int32),
        pltpu.VMEM((CH,), jnp.int32),
        pltpu.VMEM((CH,), jnp.int32),
        pltpu.VMEM((16, 128), jnp.int32),
        pltpu.VMEM((16, 128), jnp.int32),
        pltpu.VMEM((ZB,), jnp.int32),
        pltpu.VMEM_SHARED((N,), jnp.int32),
        pltpu.VMEM_SHARED((ALLOC_W,), jnp.int32),
        pltpu.SemaphoreType.DMA,
        pltpu.SemaphoreType.DMA,
    ],
)(_sc_hist)


def _fold(pt_ref, dt_ref, pa_ref, pb_ref, da_ref, db_ref, wl_ref, wr_ref,
          wca_ref, wcb_ref, wda_ref, wdb_ref, wp_ref, wd_ref):
    wl = wl_ref[:]
    wr = wr_ref[:]
    wlp = wl[0:32, :]
    wld = wl[32:64, :]
    wca_ref[:] = jnp.dot(pa_ref[:], wlp, preferred_element_type=jnp.float32)
    wcb_ref[:] = jnp.dot(pb_ref[:], wlp, preferred_element_type=jnp.float32)
    wda_ref[:] = jnp.dot(da_ref[:], wld, preferred_element_type=jnp.float32)
    wdb_ref[:] = jnp.dot(db_ref[:], wld, preferred_element_type=jnp.float32)
    wp_ref[:] = jnp.dot(pt_ref[:], wr[0:32, :],
                        preferred_element_type=jnp.float32)
    wd_ref[:] = jnp.dot(dt_ref[:], wr[32:64, :],
                        preferred_element_type=jnp.float32)


def _fold_call(pt72, dt72, pa, pb, da, db, W_l, W_r):
    h = jax.ShapeDtypeStruct((36, 128), jnp.float32)
    w = jax.ShapeDtypeStruct((72, 128), jnp.float32)
    return pl.pallas_call(
        _fold, out_shape=[h, h, h, h, w, w],
    )(pt72, dt72, pa, pb, da, db, W_l, W_r)


def _main(cnt_ref, p_ref, d_ref, bi_ref, wca_ref, wcb_ref, wda_ref, wdb_ref,
          wp_ref, wd_ref, bt_ref, b_ref, o_ref):
    w0 = cnt_ref[0]
    w1 = cnt_ref[1]
    m8 = 255
    b0 = (lax.bitwise_and(w0, m8)
          + lax.bitwise_and(w1, m8)).astype(jnp.float32)
    b1 = (lax.bitwise_and(lax.shift_right_logical(w0, 8), m8)
          + lax.bitwise_and(lax.shift_right_logical(w1, 8), m8)
          ).astype(jnp.float32)
    b2 = (lax.bitwise_and(lax.shift_right_logical(w0, 16), m8)
          + lax.bitwise_and(lax.shift_right_logical(w1, 16), m8)
          ).astype(jnp.float32)
    b3 = (lax.shift_right_logical(w0, 24)
          + lax.shift_right_logical(w1, 24)).astype(jnp.float32)
    deg = (jnp.sum(b0, axis=1, keepdims=True)
           + jnp.sum(b1, axis=1, keepdims=True))
    inv = 1.0 / jnp.maximum(deg, 1.0)
    iot = lax.broadcasted_iota(jnp.int32, (BN, 72), 1)
    ohp = (p_ref[:] == iot).astype(jnp.float32)
    ohd = (d_ref[:] == iot).astype(jnp.float32)
    ohb = (bi_ref[:] == iot).astype(jnp.float32)
    left = (jnp.dot(b0 * inv, wca_ref[:], preferred_element_type=jnp.float32)
            + jnp.dot(b1 * inv, wcb_ref[:], preferred_element_type=jnp.float32)
            + jnp.dot(b2 * inv, wda_ref[:], preferred_element_type=jnp.float32)
            + jnp.dot(b3 * inv, wdb_ref[:], preferred_element_type=jnp.float32)
            + jnp.dot(ohp, wp_ref[:], preferred_element_type=jnp.float32)
            + jnp.dot(ohd, wd_ref[:], preferred_element_type=jnp.float32)
            + b_ref[:])
    ss = jnp.sum(left * left, axis=1, keepdims=True)
    left = left * jnp.minimum(lax.rsqrt(ss), 1e12)
    left = jnp.where(left > 0, left, 0.2 * left)
    o_ref[:, 0:128] = left
    o_ref[:, 128:160] = jnp.dot(ohb, bt_ref[:],
                                preferred_element_type=jnp.float32)


def _main_call(counts, p2, d2, b2, wca, wcb, wda, wdb, wp, wd, bt72, bias):
    full = lambda shp: pl.BlockSpec(shp, lambda i: (0, 0))
    return pl.pallas_call(
        _main,
        grid=(GRID,),
        in_specs=[
            pl.BlockSpec((2, BN, RW), lambda i: (0, i, 0)),
            pl.BlockSpec((BN, 1), lambda i: (i, 0)),
            pl.BlockSpec((BN, 1), lambda i: (i, 0)),
            pl.BlockSpec((BN, 1), lambda i: (i, 0)),
            full((36, 128)),
            full((36, 128)),
            full((36, 128)),
            full((36, 128)),
            full((72, 128)),
            full((72, 128)),
            full((72, 32)),
            full((1, 128)),
        ],
        out_specs=pl.BlockSpec((BN, 160), lambda i: (i, 0)),
        out_shape=jax.ShapeDtypeStruct((N, 160), jnp.float32),
    )(counts, p2, d2, b2, wca, wcb, wda, wdb, wp, wd, bt72, bias)


def _pad_ids(a):
    return jnp.pad(a, (0, NP - N)).reshape(NP, 1)


def kernel(x, beat_info, edge_index, pitch_table, beat_table, dur_table,
           W_l, W_r, b):
    pitch = x[:, 2]
    dur = x[:, 3]
    code = pitch * 128 + dur
    ei2 = edge_index.reshape(2 * E)

    counts = _sc_hist_call(ei2, code).reshape(2, NP, RW)

    p2 = _pad_ids(pitch)
    d2 = _pad_ids(dur)
    bi2 = _pad_ids(beat_info)
    pt72 = jnp.pad(pitch_table[:66], ((0, 6), (0, 0)))
    dt72 = jnp.pad(dur_table, ((0, 6), (0, 0)))
    bt72 = jnp.pad(beat_table, ((0, 6), (0, 0)))
    pa = pitch_table[0:36]
    pb = jnp.pad(pitch_table[36:66], ((0, 6), (0, 0)))
    da = dur_table[0:36]
    db = jnp.pad(dur_table[36:66], ((0, 6), (0, 0)))
    wca, wcb, wda, wdb, wp, wd = _fold_call(pt72, dt72, pa, pb, da, db,
                                            W_l, W_r)

    return _main_call(counts, p2, d2, bi2, wca, wcb, wda, wdb, wp, wd, bt72,
                      b.reshape(1, 128))


# speedup vs baseline: 1.1119x; 1.1100x over previous
import functools

import jax
import jax.numpy as jnp
from jax import lax
from jax.experimental import pallas as pl
from jax.experimental.pallas import tpu as pltpu
from jax.experimental.pallas import tpu_sc as plsc

N = 50000
E = 800000
RW = 36
NP = 50176
ER = E // 128
ERH = ER // 2
CH = 1024
NCH = 25
PER_SC = NP * RW
PER_TILE = PER_SC // 16
ALLOC_W = 1812480
PER_TILE_Z = ALLOC_W // 16
ZB = 1920
CR = CH // 128
BN = 1024
GRID = NP // BN


def _sc_hist(ei_hbm, code_hbm, out_hbm,
             srcv, dstv, codes_v, addr_v, val_v, zbuf,
             code_sh, counts_sh, sem_g, sem_a):
    c = lax.axis_index("c")
    s = lax.axis_index("s")

    def zfill(i, _):
        zbuf[pl.ds(i * 16, 16)] = jnp.zeros((16,), jnp.int32)
        return 0
    lax.fori_loop(0, ZB // 16, zfill, 0)

    zbase = s * PER_TILE_Z
    def zcp(i, _):
        pltpu.sync_copy(zbuf, counts_sh.at[pl.ds(zbase + i * ZB, ZB)])
        return 0
    lax.fori_loop(0, PER_TILE_Z // ZB, zcp, 0)

    @pl.when(s == 0)
    def _():
        pltpu.sync_copy(code_hbm, code_sh)
    plsc.subcore_barrier()

    def _drain_adds():
        for j in range(16):
            pltpu.make_async_copy(val_v.at[j], counts_sh.at[addr_v.at[j]],
                                  sem_a).wait()

    rstart = c * ERH + (s * ERH) // 16
    rend = c * ERH + ((s + 1) * ERH) // 16
    def chunk(g, _):
        rb = rstart + g * CR
        rb_c = jnp.minimum(rb, ER - CR)
        oc = rb_c * 128
        pltpu.sync_copy(ei_hbm.at[pl.ds(oc, CH)], srcv)
        pltpu.sync_copy(ei_hbm.at[pl.ds(E + oc, CH)], dstv)
        gcps = [pltpu.async_copy(code_sh.at[srcv.at[pl.ds(j * 128, 128)]],
                                 codes_v.at[pl.ds(j * 128, 128)], sem_g)
                for j in range(CR)]
        for cp in gcps:
            cp.wait()
        @pl.when(g > 0)
        def _():
            _drain_adds()
        for i in range(CH // 16):
            j, k = i // 8, (i % 8) * 16
            rg = rb_c + j
            pen = jnp.where((rg >= rb) & (rg < rend), 0, 10000000)
            cv = codes_v[pl.ds(j * 128 + k, 16)]
            dv = dstv[pl.ds(j * 128 + k, 16)]
            p = lax.shift_right_logical(cv, 7)
            dr = lax.bitwise_and(cv, 127)
            rowt = jnp.minimum(dv + pen, NP)
            base = rowt * RW
            pin = p < 36
            din = dr < 36
            addr_v[j, pl.ds(k, 16)] = base + jnp.where(pin, p, p - 36)
            addr_v[j + 8, pl.ds(k, 16)] = base + jnp.where(din, dr, dr - 36)
            val_v[j, pl.ds(k, 16)] = jnp.where(pin, 1, 256)
            val_v[j + 8, pl.ds(k, 16)] = jnp.where(din, 65536, 16777216)
        for j in range(16):
            pltpu.async_copy(val_v.at[j], counts_sh.at[addr_v.at[j]], sem_a,
                             add=True)
        return 0
    lax.fori_loop(0, NCH, chunk, 0)
    _drain_adds()
    plsc.subcore_barrier()

    gout = c * PER_SC + s * PER_TILE
    pltpu.sync_copy(counts_sh.at[pl.ds(s * PER_TILE, PER_TILE)],
                    out_hbm.at[pl.ds(gout, PER_TILE)])


_sc_hist_call = functools.partial(
    pl.kernel,
    mesh=plsc.VectorSubcoreMesh(core_axis_name="c", subcore_axis_name="s"),
    compiler_params=pltpu.CompilerParams(needs_layout_passes=False),
    out_type=jax.ShapeDtypeStruct((2 * NP * RW,), jnp.int32),
    scratch_types=[
        pltpu.VMEM((CH,), jnp.int32),
        pltpu.VMEM((CH,), jnp.int32),
        pltpu.VMEM((CH,), jnp.int32),
        pltpu.VMEM((16, 128), jnp.int32),
        pltpu.VMEM((16, 128), jnp.int32),
        pltpu.VMEM((ZB,), jnp.int32),
        pltpu.VMEM_SHARED((N,), jnp.int32),
        pltpu.VMEM_SHARED((ALLOC_W,), jnp.int32),
        pltpu.SemaphoreType.DMA,
        pltpu.SemaphoreType.DMA,
    ],
)(_sc_hist)


def _fold(pt_ref, dt_ref, pa_ref, pb_ref, da_ref, db_ref, wl_ref, wr_ref,
          wca_ref, wcb_ref, wda_ref, wdb_ref, wp_ref, wd_ref):
    wl = wl_ref[:]
    wr = wr_ref[:]
    wlp = wl[0:32, :]
    wld = wl[32:64, :]
    wca_ref[:] = jnp.dot(pa_ref[:], wlp, preferred_element_type=jnp.float32)
    wcb_ref[:] = jnp.dot(pb_ref[:], wlp, preferred_element_type=jnp.float32)
    wda_ref[:] = jnp.dot(da_ref[:], wld, preferred_element_type=jnp.float32)
    wdb_ref[:] = jnp.dot(db_ref[:], wld, preferred_element_type=jnp.float32)
    wp_ref[:] = jnp.dot(pt_ref[:], wr[0:32, :],
                        preferred_element_type=jnp.float32)
    wd_ref[:] = jnp.dot(dt_ref[:], wr[32:64, :],
                        preferred_element_type=jnp.float32)


def _fold_call(pt72, dt72, pa, pb, da, db, W_l, W_r):
    h = jax.ShapeDtypeStruct((36, 128), jnp.float32)
    w = jax.ShapeDtypeStruct((72, 128), jnp.float32)
    return pl.pallas_call(
        _fold, out_shape=[h, h, h, h, w, w],
    )(pt72, dt72, pa, pb, da, db, W_l, W_r)


def _main(cnt_ref, p_ref, d_ref, bi_ref, wca_ref, wcb_ref, wda_ref, wdb_ref,
          wp_ref, wd_ref, bt_ref, b_ref, o_ref):
    w0 = cnt_ref[0]
    w1 = cnt_ref[1]
    m8 = 255
    b0 = (lax.bitwise_and(w0, m8)
          + lax.bitwise_and(w1, m8)).astype(jnp.float32)
    b1 = (lax.bitwise_and(lax.shift_right_logical(w0, 8), m8)
          + lax.bitwise_and(lax.shift_right_logical(w1, 8), m8)
          ).astype(jnp.float32)
    b2 = (lax.bitwise_and(lax.shift_right_logical(w0, 16), m8)
          + lax.bitwise_and(lax.shift_right_logical(w1, 16), m8)
          ).astype(jnp.float32)
    b3 = (lax.shift_right_logical(w0, 24)
          + lax.shift_right_logical(w1, 24)).astype(jnp.float32)
    deg = (jnp.sum(b0, axis=1, keepdims=True)
           + jnp.sum(b1, axis=1, keepdims=True))
    inv = 1.0 / jnp.maximum(deg, 1.0)
    iot = lax.broadcasted_iota(jnp.int32, (BN, 72), 1)
    ohp = (p_ref[:] == iot).astype(jnp.float32)
    ohd = (d_ref[:] == iot).astype(jnp.float32)
    ohb = (bi_ref[:] == iot).astype(jnp.float32)
    left = (jnp.dot(b0 * inv, wca_ref[:], preferred_element_type=jnp.float32)
            + jnp.dot(b1 * inv, wcb_ref[:], preferred_element_type=jnp.float32)
            + jnp.dot(b2 * inv, wda_ref[:], preferred_element_type=jnp.float32)
            + jnp.dot(b3 * inv, wdb_ref[:], preferred_element_type=jnp.float32)
            + jnp.dot(ohp, wp_ref[:], preferred_element_type=jnp.float32)
            + jnp.dot(ohd, wd_ref[:], preferred_element_type=jnp.float32)
            + b_ref[:])
    ss = jnp.sum(left * left, axis=1, keepdims=True)
    left = left * jnp.minimum(lax.rsqrt(ss), 1e12)
    left = jnp.where(left > 0, left, 0.2 * left)
    o_ref[:, 0:128] = left
    o_ref[:, 128:160] = jnp.dot(ohb, bt_ref[:],
                                preferred_element_type=jnp.float32)


def _main_call(counts, p2, d2, b2, wca, wcb, wda, wdb, wp, wd, bt72, bias):
    full = lambda shp: pl.BlockSpec(shp, lambda i: (0, 0))
    return pl.pallas_call(
        _main,
        grid=(GRID,),
        in_specs=[
            pl.BlockSpec((2, BN, RW), lambda i: (0, i, 0)),
            pl.BlockSpec((BN, 1), lambda i: (i, 0)),
            pl.BlockSpec((BN, 1), lambda i: (i, 0)),
            pl.BlockSpec((BN, 1), lambda i: (i, 0)),
            full((36, 128)),
            full((36, 128)),
            full((36, 128)),
            full((36, 128)),
            full((72, 128)),
            full((72, 128)),
            full((72, 32)),
            full((1, 128)),
        ],
        out_specs=pl.BlockSpec((BN, 160), lambda i: (i, 0)),
        out_shape=jax.ShapeDtypeStruct((N, 160), jnp.float32),
    )(counts, p2, d2, b2, wca, wcb, wda, wdb, wp, wd, bt72, bias)


def _pad_ids(a):
    return jnp.pad(a, (0, NP - N)).reshape(NP, 1)


def kernel(x, beat_info, edge_index, pitch_table, beat_table, dur_table,
           W_l, W_r, b):
    pitch = x[:, 2]
    dur = x[:, 3]
    code = pitch * 128 + dur
    ei2 = edge_index.reshape(2 * E)

    counts = _sc_hist_call(ei2, code).reshape(2, NP, RW)

    p2 = _pad_ids(pitch)
    d2 = _pad_ids(dur)
    bi2 = _pad_ids(beat_info)
    pt72 = jnp.pad(pitch_table[:66], ((0, 6), (0, 0)))
    dt72 = jnp.pad(dur_table, ((0, 6), (0, 0)))
    bt72 = jnp.pad(beat_table, ((0, 6), (0, 0)))
    pa = pitch_table[0:36]
    pb = jnp.pad(pitch_table[36:66], ((0, 6), (0, 0)))
    da = dur_table[0:36]
    db = jnp.pad(dur_table[36:66], ((0, 6), (0, 0)))
    wca, wcb, wda, wdb, wp, wd = _fold_call(pt72, dt72, pa, pb, da, db,
                                            W_l, W_r)

    return _main_call(counts, p2, d2, bi2, wca, wcb, wda, wdb, wp, wd, bt72,
                      b.reshape(1, 128))


# speedup vs baseline: 1.1799x; 1.0612x over previous
import functools

import jax
import jax.numpy as jnp
from jax import lax
from jax.experimental import pallas as pl
from jax.experimental.pallas import tpu as pltpu
from jax.experimental.pallas import tpu_sc as plsc

N = 50000
E = 800000
RW = 36
NP = 50176
ER = E // 128
ERH = ER // 2
CH = 1024
NCH = 25
PER_SC = NP * RW
PER_TILE = PER_SC // 16
ALLOC_W = 1812480
PER_TILE_Z = ALLOC_W // 16
ZB = 1920
CR = CH // 128
BN = 3136
GRID = NP // BN


def _sc_hist(ei_hbm, code_hbm, out_hbm,
             srcv, dstv, codes_v, addr_v, val_v, zbuf,
             code_sh, counts_sh, sem_g, sem_a):
    c = lax.axis_index("c")
    s = lax.axis_index("s")

    def zfill(i, _):
        zbuf[pl.ds(i * 16, 16)] = jnp.zeros((16,), jnp.int32)
        return 0
    lax.fori_loop(0, ZB // 16, zfill, 0)

    zbase = s * PER_TILE_Z
    def zcp(i, _):
        pltpu.sync_copy(zbuf, counts_sh.at[pl.ds(zbase + i * ZB, ZB)])
        return 0
    lax.fori_loop(0, PER_TILE_Z // ZB, zcp, 0)

    @pl.when(s == 0)
    def _():
        pltpu.sync_copy(code_hbm, code_sh)
    plsc.subcore_barrier()

    def _drain_adds():
        for j in range(16):
            pltpu.make_async_copy(val_v.at[j], counts_sh.at[addr_v.at[j]],
                                  sem_a).wait()

    rstart = c * ERH + (s * ERH) // 16
    rend = c * ERH + ((s + 1) * ERH) // 16
    def chunk(g, _):
        rb = rstart + g * CR
        rb_c = jnp.minimum(rb, ER - CR)
        oc = rb_c * 128
        pltpu.sync_copy(ei_hbm.at[pl.ds(oc, CH)], srcv)
        pltpu.sync_copy(ei_hbm.at[pl.ds(E + oc, CH)], dstv)
        gcps = [pltpu.async_copy(code_sh.at[srcv.at[pl.ds(j * 128, 128)]],
                                 codes_v.at[pl.ds(j * 128, 128)], sem_g)
                for j in range(CR)]
        for cp in gcps:
            cp.wait()
        @pl.when(g > 0)
        def _():
            _drain_adds()
        for i in range(CH // 16):
            j, k = i // 8, (i % 8) * 16
            rg = rb_c + j
            pen = jnp.where((rg >= rb) & (rg < rend), 0, 10000000)
            cv = codes_v[pl.ds(j * 128 + k, 16)]
            dv = dstv[pl.ds(j * 128 + k, 16)]
            p = lax.shift_right_logical(cv, 7)
            dr = lax.bitwise_and(cv, 127)
            rowt = jnp.minimum(dv + pen, NP)
            base = rowt * RW
            pin = p < 36
            din = dr < 36
            addr_v[j, pl.ds(k, 16)] = base + jnp.where(pin, p, p - 36)
            addr_v[j + 8, pl.ds(k, 16)] = base + jnp.where(din, dr, dr - 36)
            val_v[j, pl.ds(k, 16)] = jnp.where(pin, 1, 256)
            val_v[j + 8, pl.ds(k, 16)] = jnp.where(din, 65536, 16777216)
        for j in range(16):
            pltpu.async_copy(val_v.at[j], counts_sh.at[addr_v.at[j]], sem_a,
                             add=True)
        return 0
    lax.fori_loop(0, NCH, chunk, 0)
    _drain_adds()
    plsc.subcore_barrier()

    gout = c * PER_SC + s * PER_TILE
    pltpu.sync_copy(counts_sh.at[pl.ds(s * PER_TILE, PER_TILE)],
                    out_hbm.at[pl.ds(gout, PER_TILE)])


_sc_hist_call = functools.partial(
    pl.kernel,
    mesh=plsc.VectorSubcoreMesh(core_axis_name="c", subcore_axis_name="s"),
    compiler_params=pltpu.CompilerParams(needs_layout_passes=False),
    out_type=jax.ShapeDtypeStruct((2 * NP * RW,), jnp.int32),
    scratch_types=[
        pltpu.VMEM((CH,), jnp.int32),
        pltpu.VMEM((CH,), jnp.int32),
        pltpu.VMEM((CH,), jnp.int32),
        pltpu.VMEM((16, 128), jnp.int32),
        pltpu.VMEM((16, 128), jnp.int32),
        pltpu.VMEM((ZB,), jnp.int32),
        pltpu.VMEM_SHARED((N,), jnp.int32),
        pltpu.VMEM_SHARED((ALLOC_W,), jnp.int32),
        pltpu.SemaphoreType.DMA,
        pltpu.SemaphoreType.DMA,
    ],
)(_sc_hist)


def _fold(pt_ref, dt_ref, pa_ref, pb_ref, da_ref, db_ref, wl_ref, wr_ref,
          wca_ref, wcb_ref, wda_ref, wdb_ref, wp_ref, wd_ref):
    wl = wl_ref[:]
    wr = wr_ref[:]
    wlp = wl[0:32, :]
    wld = wl[32:64, :]
    wca_ref[:] = jnp.dot(pa_ref[:], wlp, preferred_element_type=jnp.float32)
    wcb_ref[:] = jnp.dot(pb_ref[:], wlp, preferred_element_type=jnp.float32)
    wda_ref[:] = jnp.dot(da_ref[:], wld, preferred_element_type=jnp.float32)
    wdb_ref[:] = jnp.dot(db_ref[:], wld, preferred_element_type=jnp.float32)
    wp_ref[:] = jnp.dot(pt_ref[:], wr[0:32, :],
                        preferred_element_type=jnp.float32)
    wd_ref[:] = jnp.dot(dt_ref[:], wr[32:64, :],
                        preferred_element_type=jnp.float32)


def _fold_call(pt72, dt72, pa, pb, da, db, W_l, W_r):
    h = jax.ShapeDtypeStruct((36, 128), jnp.float32)
    w = jax.ShapeDtypeStruct((72, 128), jnp.float32)
    return pl.pallas_call(
        _fold, out_shape=[h, h, h, h, w, w],
    )(pt72, dt72, pa, pb, da, db, W_l, W_r)


def _main(cnt_ref, p_ref, d_ref, bi_ref, wca_ref, wcb_ref, wda_ref, wdb_ref,
          wp_ref, wd_ref, bt_ref, b_ref, o_ref):
    w0 = cnt_ref[0]
    w1 = cnt_ref[1]
    m8 = 255
    b0 = (lax.bitwise_and(w0, m8)
          + lax.bitwise_and(w1, m8)).astype(jnp.float32)
    b1 = (lax.bitwise_and(lax.shift_right_logical(w0, 8), m8)
          + lax.bitwise_and(lax.shift_right_logical(w1, 8), m8)
          ).astype(jnp.float32)
    b2 = (lax.bitwise_and(lax.shift_right_logical(w0, 16), m8)
          + lax.bitwise_and(lax.shift_right_logical(w1, 16), m8)
          ).astype(jnp.float32)
    b3 = (lax.shift_right_logical(w0, 24)
          + lax.shift_right_logical(w1, 24)).astype(jnp.float32)
    deg = (jnp.sum(b0, axis=1, keepdims=True)
           + jnp.sum(b1, axis=1, keepdims=True))
    inv = 1.0 / jnp.maximum(deg, 1.0)
    iot = lax.broadcasted_iota(jnp.int32, (BN, 72), 1)
    ohp = (p_ref[:] == iot).astype(jnp.float32)
    ohd = (d_ref[:] == iot).astype(jnp.float32)
    ohb = (bi_ref[:] == iot).astype(jnp.float32)
    left = (jnp.dot(b0 * inv, wca_ref[:], preferred_element_type=jnp.float32)
            + jnp.dot(b1 * inv, wcb_ref[:], preferred_element_type=jnp.float32)
            + jnp.dot(b2 * inv, wda_ref[:], preferred_element_type=jnp.float32)
            + jnp.dot(b3 * inv, wdb_ref[:], preferred_element_type=jnp.float32)
            + jnp.dot(ohp, wp_ref[:], preferred_element_type=jnp.float32)
            + jnp.dot(ohd, wd_ref[:], preferred_element_type=jnp.float32)
            + b_ref[:])
    ss = jnp.sum(left * left, axis=1, keepdims=True)
    left = left * jnp.minimum(lax.rsqrt(ss), 1e12)
    left = jnp.where(left > 0, left, 0.2 * left)
    o_ref[:, 0:128] = left
    o_ref[:, 128:160] = jnp.dot(ohb, bt_ref[:],
                                preferred_element_type=jnp.float32)


def _main_call(counts, p2, d2, b2, wca, wcb, wda, wdb, wp, wd, bt72, bias):
    full = lambda shp: pl.BlockSpec(shp, lambda i: (0, 0))
    return pl.pallas_call(
        _main,
        grid=(GRID,),
        in_specs=[
            pl.BlockSpec((2, BN, RW), lambda i: (0, i, 0)),
            pl.BlockSpec((BN, 1), lambda i: (i, 0)),
            pl.BlockSpec((BN, 1), lambda i: (i, 0)),
            pl.BlockSpec((BN, 1), lambda i: (i, 0)),
            full((36, 128)),
            full((36, 128)),
            full((36, 128)),
            full((36, 128)),
            full((72, 128)),
            full((72, 128)),
            full((72, 32)),
            full((1, 128)),
        ],
        out_specs=pl.BlockSpec((BN, 160), lambda i: (i, 0)),
        out_shape=jax.ShapeDtypeStruct((N, 160), jnp.float32),
    )(counts, p2, d2, b2, wca, wcb, wda, wdb, wp, wd, bt72, bias)


def _pad_ids(a):
    return jnp.pad(a, (0, NP - N)).reshape(NP, 1)


def kernel(x, beat_info, edge_index, pitch_table, beat_table, dur_table,
           W_l, W_r, b):
    pitch = x[:, 2]
    dur = x[:, 3]
    code = pitch * 128 + dur
    ei2 = edge_index.reshape(2 * E)

    counts = _sc_hist_call(ei2, code).reshape(2, NP, RW)

    p2 = _pad_ids(pitch)
    d2 = _pad_ids(dur)
    bi2 = _pad_ids(beat_info)
    pt72 = jnp.pad(pitch_table[:66], ((0, 6), (0, 0)))
    dt72 = jnp.pad(dur_table, ((0, 6), (0, 0)))
    bt72 = jnp.pad(beat_table, ((0, 6), (0, 0)))
    pa = pitch_table[0:36]
    pb = jnp.pad(pitch_table[36:66], ((0, 6), (0, 0)))
    da = dur_table[0:36]
    db = jnp.pad(dur_table[36:66], ((0, 6), (0, 0)))
    wca, wcb, wda, wdb, wp, wd = _fold_call(pt72, dt72, pa, pb, da, db,
                                            W_l, W_r)

    return _main_call(counts, p2, d2, bi2, wca, wcb, wda, wdb, wp, wd, bt72,
                      b.reshape(1, 128))
